# baseline (speedup 1.0000x reference)
import functools

import jax
import jax.numpy as jnp
from jax import lax
from jax.experimental import pallas as pl
from jax.experimental.pallas import tpu as pltpu
from jax.experimental.pallas import tpu_sc as plsc

NR = 10000
NG = 10000
RE = 128
HD = RE // 2
NL = 2
E = 160000

NC = 2
NS = 16
NW = NC * NS
PER_W = E // NW
K = 40
NCH = PER_W // K
ROWS_A = 624
ROWS_LAST = NR - (NS - 1) * ROWS_A

_F32 = jnp.float32


def _pre_body(xg_ref, wb_ref, wv_ref, bmv0_ref, bmv1_ref):
    xg = xg_ref[...]
    bmv0_ref[:, :RE] = jnp.dot(xg, wb_ref[0], preferred_element_type=_F32)
    bmv0_ref[:, RE:] = jnp.dot(xg, wv_ref[0], preferred_element_type=_F32)
    bmv1_ref[:, :RE] = jnp.dot(xg, wb_ref[1], preferred_element_type=_F32)
    bmv1_ref[:, RE:] = jnp.dot(xg, wv_ref[1], preferred_element_type=_F32)


def _dense0_body(xr_ref, wa_ref, wm_ref, ws_ref, wself_ref, a_ref, m_ref, sc_ref):
    curr = xr_ref[...]
    a_ref[...] = jnp.dot(curr, wa_ref[...], preferred_element_type=_F32)
    m_ref[...] = jnp.dot(curr, wm_ref[...], preferred_element_type=_F32)
    sc_ref[...] = jnp.dot(curr, ws_ref[...] + wself_ref[...],
                          preferred_element_type=_F32)


def _norm(x, g, b):
    x = jax.nn.gelu(x)
    mu = jnp.mean(x, axis=-1, keepdims=True)
    d = x - mu
    var = jnp.mean(d * d, axis=-1, keepdims=True)
    return d * lax.rsqrt(var + 1e-5) * g + b


def _fuse_body(sc_ref, part_ref, g_ref, b_ref, wa_ref, wm_ref, ws_ref,
               wself_ref, a_ref, m_ref, scout_ref):
    x = sc_ref[...] + part_ref[0] + part_ref[1]
    curr = _norm(x, g_ref[...], b_ref[...])
    a_ref[...] = jnp.dot(curr, wa_ref[...], preferred_element_type=_F32)
    m_ref[...] = jnp.dot(curr, wm_ref[...], preferred_element_type=_F32)
    scout_ref[...] = jnp.dot(curr, ws_ref[...] + wself_ref[...],
                             preferred_element_type=_F32)


def _head_body(sc_ref, part_ref, g_ref, b_ref, w1_ref, b1_ref, w2_ref, out_ref):
    x = sc_ref[...] + part_ref[0] + part_ref[1]
    curr = _norm(x, g_ref[...], b_ref[...])
    w1 = w1_ref[...]
    b1 = b1_ref[...]
    s1 = jnp.tanh(jnp.dot(curr, w1, preferred_element_type=_F32) + b1)
    swapped = jnp.concatenate([curr[:, HD:], curr[:, :HD]], axis=-1)
    s2 = jnp.tanh(jnp.dot(swapped, w1, preferred_element_type=_F32) + b1)
    w2v = w2_ref[...][:, 0]
    out_ref[...] = jnp.sum((s1 - s2) * w2v, axis=-1, keepdims=True)


def _sds(shape):
    return jax.ShapeDtypeStruct(shape, _F32)


def _make_edge_kernel():
    mesh = plsc.VectorSubcoreMesh(core_axis_name="c", subcore_axis_name="s")

    @functools.partial(
        pl.kernel,
        mesh=mesh,
        out_type=_sds((NC, NR, RE)),
        scratch_types=[
            pltpu.VMEM((3, K), jnp.int32),
            pltpu.VMEM((3, K), jnp.int32),
            pltpu.VMEM((2, K, RE), _F32),
            pltpu.VMEM((2, K, 2 * RE), _F32),
            pltpu.VMEM((2, K, RE), _F32),
            pltpu.VMEM_SHARED((NR, RE), _F32),
            pltpu.SemaphoreType.DMA((3,)),
            pltpu.SemaphoreType.DMA((2,)),
        ],
    )
    def edge_kernel(a_hbm, bmv_hbm, m_hbm, srcg_hbm, dstg_hbm,
                    srcr_hbm, dstr_hbm, zeros_hbm, out_hbm,
                    src_i, dst_i, rows_a, rows_bv, msg, acc,
                    sem_i, sem_g):
        c = lax.axis_index("c")
        s = lax.axis_index("s")
        w = c * NS + s
        row0 = s * ROWS_A
        @pl.when(s < NS - 1)
        def _():
            pltpu.sync_copy(zeros_hbm.at[pl.ds(0, ROWS_A)],
                            acc.at[pl.ds(row0, ROWS_A)])

        @pl.when(s == NS - 1)
        def _():
            pltpu.sync_copy(zeros_hbm, acc.at[pl.ds(row0, ROWS_LAST)])

        plsc.subcore_barrier()
        base_w = w * PER_W

        def run_phase(src_hbm, dst_hbm, gather_starts, gather_waits, compute):
            def idx_start(i, b):
                base = base_w + i * K
                pltpu.async_copy(src_hbm.at[pl.ds(base, K)], src_i.at[b],
                                 sem_i.at[b])
                pltpu.async_copy(dst_hbm.at[pl.ds(base, K)], dst_i.at[b],
                                 sem_i.at[b])

            def idx_wait(i, b):
                base = base_w + i * K
                pltpu.make_async_copy(src_hbm.at[pl.ds(base, K)],
                                      src_i.at[b], sem_i.at[b]).wait()
                pltpu.make_async_copy(dst_hbm.at[pl.ds(base, K)],
                                      dst_i.at[b], sem_i.at[b]).wait()

            def step(c, i2, i1, o, s, d, start2, start1):
                if start2:
                    idx_start(c + 2, i2)
                if start1:
                    idx_wait(c + 1, i1)
                    gather_starts(i1, o)
                gather_waits(d, s)
                if compute is not None:
                    compute(s)
                pltpu.sync_copy(msg.at[s], acc.at[dst_i.at[d]], add=True)

            idx_start(0, 0)
            idx_start(1, 1)
            idx_wait(0, 0)
            gather_starts(0, 0)

            def six(t, carry):
                c0 = 6 * t
                for j in range(6):
                    step(c0 + j, (j + 2) % 3, (j + 1) % 3, (j + 1) % 2,
                         j % 2, j % 3, True, True)
                return carry

            lax.fori_loop(0, NCH // 6, six, 0, unroll=False)
            for c in range(6 * (NCH // 6), NCH):
                step(c, (c + 2) % 3, (c + 1) % 3, (c + 1) % 2, c % 2,
                     c % 3, c + 2 < NCH, c + 1 < NCH)

        def g2r_starts(i, b):
            pltpu.async_copy(bmv_hbm.at[src_i.at[i]], rows_bv.at[b],
                             sem_g.at[b])
            pltpu.async_copy(a_hbm.at[dst_i.at[i]], rows_a.at[b],
                             sem_g.at[b])

        def g2r_waits(i, b):
            pltpu.make_async_copy(bmv_hbm.at[src_i.at[i]], rows_bv.at[b],
                                  sem_g.at[b]).wait()
            pltpu.make_async_copy(a_hbm.at[dst_i.at[i]], rows_a.at[b],
                                  sem_g.at[b]).wait()

        def g2r_compute(b):
            def edge_e(e, c2):
                for cc in range(RE // 16):
                    sl = pl.ds(cc * 16, 16)
                    sl2 = pl.ds(RE + cc * 16, 16)
                    t = rows_a[b, e, sl] + rows_bv[b, e, sl]
                    sig = 1.0 / (1.0 + jnp.exp(-t))
                    msg[b, e, sl] = rows_bv[b, e, sl2] * sig
                return c2

            lax.fori_loop(0, K, edge_e, 0, unroll=False)

        run_phase(srcg_hbm, dstg_hbm, g2r_starts, g2r_waits, g2r_compute)

        def r2r_starts(i, b):
            pltpu.async_copy(m_hbm.at[src_i.at[i]], msg.at[b], sem_g.at[b])

        def r2r_waits(i, b):
            pltpu.make_async_copy(m_hbm.at[src_i.at[i]], msg.at[b],
                                  sem_g.at[b]).wait()

        run_phase(srcr_hbm, dstr_hbm, r2r_starts, r2r_waits, None)

        plsc.subcore_barrier()

        @pl.when(s < NS - 1)
        def _():
            pltpu.sync_copy(acc.at[pl.ds(row0, ROWS_A)],
                            out_hbm.at[c, pl.ds(row0, ROWS_A)])

        @pl.when(s == NS - 1)
        def _():
            pltpu.sync_copy(acc.at[pl.ds(row0, ROWS_LAST)],
                            out_hbm.at[c, pl.ds(row0, ROWS_LAST)])

    return edge_kernel


_edge_kernel = _make_edge_kernel()


def kernel(x_G, x_R, edge_index_G_to_R, edge_index_R_to_R, W_A, W_B, W_V,
           W_S, W_self, W_msg, ln_g, ln_b, W1, b1, w2):
    xg = x_G[0]
    xr = x_R[0]
    srcg = edge_index_G_to_R[0].astype(jnp.int32)
    dstg = edge_index_G_to_R[1].astype(jnp.int32)
    srcr = edge_index_R_to_R[0].astype(jnp.int32)
    dstr = edge_index_R_to_R[1].astype(jnp.int32)
    zeros = jnp.zeros((ROWS_LAST, RE), _F32)

    bmv0, bmv1 = pl.pallas_call(
        _pre_body,
        out_shape=[_sds((NG, 2 * RE))] * 2,
    )(xg, W_B, W_V)

    a0, m0, sc0 = pl.pallas_call(
        _dense0_body,
        out_shape=[_sds((NR, RE))] * 3,
    )(xr, W_A[0], W_msg[0], W_S[0], W_self[0])

    part0 = _edge_kernel(a0, bmv0, m0, srcg, dstg, srcr, dstr, zeros)

    a1, m1, sc1 = pl.pallas_call(
        _fuse_body,
        out_shape=[_sds((NR, RE))] * 3,
    )(sc0, part0, ln_g[0], ln_b[0], W_A[1], W_msg[1], W_S[1], W_self[1])

    part1 = _edge_kernel(a1, bmv1, m1, srcg, dstg, srcr, dstr, zeros)

    out = pl.pallas_call(
        _head_body,
        out_shape=_sds((NR, 1)),
    )(sc1, part1, ln_g[1], ln_b[1], W1, b1, w2)

    return out.reshape(1, NR)


# speedup vs baseline: 3.9303x; 3.9303x over previous
import functools

import jax
import jax.numpy as jnp
from jax import lax
from jax.experimental import pallas as pl
from jax.experimental.pallas import tpu as pltpu
from jax.experimental.pallas import tpu_sc as plsc

NR = 10000
NG = 10000
RE = 128
HD = RE // 2
NL = 2
E = 160000

NC = 2
NS = 16
NW = NC * NS
PER_W = E // NW
K = 40
NCH = PER_W // K
ROWS_A = 624
ROWS_LAST = NR - (NS - 1) * ROWS_A

_F32 = jnp.float32


def _pre_body(xg_ref, wb_ref, wv_ref, bm0_ref, v0_ref, bm1_ref, v1_ref):
    xg = xg_ref[...]
    bm0_ref[...] = jnp.dot(xg, wb_ref[0],
                           preferred_element_type=_F32).astype(jnp.bfloat16)
    v0_ref[...] = jnp.dot(xg, wv_ref[0],
                          preferred_element_type=_F32).astype(jnp.bfloat16)
    bm1_ref[...] = jnp.dot(xg, wb_ref[1],
                           preferred_element_type=_F32).astype(jnp.bfloat16)
    v1_ref[...] = jnp.dot(xg, wv_ref[1],
                          preferred_element_type=_F32).astype(jnp.bfloat16)


def _dense0_body(xr_ref, wa_ref, wm_ref, ws_ref, wself_ref, a_ref, m_ref, sc_ref):
    curr = xr_ref[...]
    a_ref[...] = jnp.dot(curr, wa_ref[...], preferred_element_type=_F32)
    m_ref[...] = jnp.dot(curr, wm_ref[...], preferred_element_type=_F32)
    sc_ref[...] = jnp.dot(curr, ws_ref[...] + wself_ref[...],
                          preferred_element_type=_F32)


def _norm(x, g, b):
    x = jax.nn.gelu(x)
    mu = jnp.mean(x, axis=-1, keepdims=True)
    d = x - mu
    var = jnp.mean(d * d, axis=-1, keepdims=True)
    return d * lax.rsqrt(var + 1e-5) * g + b


def _fuse_body(sc_ref, part_ref, g_ref, b_ref, wa_ref, wm_ref, ws_ref,
               wself_ref, a_ref, m_ref, scout_ref):
    x = sc_ref[...] + part_ref[0] + part_ref[1]
    curr = _norm(x, g_ref[...], b_ref[...])
    a_ref[...] = jnp.dot(curr, wa_ref[...], preferred_element_type=_F32)
    m_ref[...] = jnp.dot(curr, wm_ref[...], preferred_element_type=_F32)
    scout_ref[...] = jnp.dot(curr, ws_ref[...] + wself_ref[...],
                             preferred_element_type=_F32)


def _head_body(sc_ref, part_ref, g_ref, b_ref, w1_ref, b1_ref, w2_ref, out_ref):
    x = sc_ref[...] + part_ref[0] + part_ref[1]
    curr = _norm(x, g_ref[...], b_ref[...])
    w1 = w1_ref[...]
    b1 = b1_ref[...]
    s1 = jnp.tanh(jnp.dot(curr, w1, preferred_element_type=_F32) + b1)
    swapped = jnp.concatenate([curr[:, HD:], curr[:, :HD]], axis=-1)
    s2 = jnp.tanh(jnp.dot(swapped, w1, preferred_element_type=_F32) + b1)
    w2v = w2_ref[...][:, 0]
    out_ref[...] = jnp.sum((s1 - s2) * w2v, axis=-1, keepdims=True)


def _sds(shape):
    return jax.ShapeDtypeStruct(shape, _F32)


def _make_edge_kernel():
    mesh = plsc.VectorSubcoreMesh(core_axis_name="c", subcore_axis_name="s")

    @functools.partial(
        pl.kernel,
        mesh=mesh,
        out_type=_sds((NC, NR, RE)),
        scratch_types=[
            pltpu.VMEM((3, K), jnp.int32),
            pltpu.VMEM((3, K), jnp.int32),
            pltpu.VMEM((3, K, RE), _F32),
            pltpu.VMEM((3, K, RE), _F32),
            pltpu.VMEM((3, K, RE), _F32),
            pltpu.VMEM_SHARED((NR, RE), _F32),
            pltpu.SemaphoreType.DMA((3,)),
            pltpu.SemaphoreType.DMA((3,)),
        ],
    )
    def edge_kernel(a_hbm, bmv_hbm, m_hbm, srcg_hbm, dstg_hbm,
                    srcr_hbm, dstr_hbm, zeros_hbm, out_hbm,
                    src_v, dst_v, rows_a, rows_bv, rows_v, acc,
                    sem_i, sem_g):
        c = lax.axis_index("c")
        s = lax.axis_index("s")
        w = c * NS + s
        row0 = s * ROWS_A
        @pl.when(s < NS - 1)
        def _():
            pltpu.sync_copy(zeros_hbm.at[pl.ds(0, ROWS_A)],
                            acc.at[pl.ds(row0, ROWS_A)])

        @pl.when(s == NS - 1)
        def _():
            pltpu.sync_copy(zeros_hbm, acc.at[pl.ds(row0, ROWS_LAST)])

        plsc.subcore_barrier()
        base_w = w * PER_W

        NT = (NCH - 2) // 3

        def run_phase(src_hbm, dst_hbm, gather_starts, gather_waits, compute):
            def idx_start(i, b):
                base = base_w + i * K
                pltpu.async_copy(src_hbm.at[pl.ds(base, K)], src_v.at[b],
                                 sem_i.at[b])
                pltpu.async_copy(dst_hbm.at[pl.ds(base, K)], dst_v.at[b],
                                 sem_i.at[b])

            def idx_wait(i, b):
                base = base_w + i * K
                pltpu.make_async_copy(src_hbm.at[pl.ds(base, K)],
                                      src_v.at[b], sem_i.at[b]).wait()
                pltpu.make_async_copy(dst_hbm.at[pl.ds(base, K)],
                                      dst_v.at[b], sem_i.at[b]).wait()

            def finish(c, s):
                gather_waits(s)
                if compute is not None:
                    compute(s)
                pltpu.sync_copy(rows_v.at[s], acc.at[dst_v.at[s]], add=True)

            def step(c, s, o, p, prefetch):
                if prefetch:
                    idx_start(c + 2, p)
                idx_wait(c + 1, o)
                gather_starts(o)
                finish(c, s)

            idx_start(0, 0)
            idx_start(1, 1)
            idx_wait(0, 0)
            gather_starts(0)

            def triple(t, carry):
                c0 = 3 * t
                step(c0, 0, 1, 2, True)
                step(c0 + 1, 1, 2, 0, True)
                step(c0 + 2, 2, 0, 1, True)
                return carry

            lax.fori_loop(0, NT, triple, 0, unroll=False)
            step(NCH - 2, 0, 1, 2, False)
            finish(NCH - 1, 1)

        def g2r_starts(b):
            pltpu.async_copy(bmv_hbm.at[src_v.at[b]], rows_bv.at[b],
                             sem_g.at[b])
            pltpu.async_copy(a_hbm.at[dst_v.at[b]], rows_a.at[b], sem_g.at[b])

        def g2r_waits(b):
            pltpu.make_async_copy(bmv_hbm.at[src_v.at[b]], rows_bv.at[b],
                                  sem_g.at[b]).wait()
            pltpu.make_async_copy(a_hbm.at[dst_v.at[b]], rows_a.at[b],
                                  sem_g.at[b]).wait()

        def g2r_compute(b):
            def edge_e(e, c2):
                for cc in range(RE // 16):
                    sl = pl.ds(cc * 16, 16)
                    pair = lax.bitcast_convert_type(rows_bv[b, e, sl],
                                                    jnp.int32)
                    bm16 = lax.bitcast_convert_type(pair << 16, _F32)
                    v16 = lax.bitcast_convert_type(pair & jnp.int32(-65536),
                                                   _F32)
                    t = rows_a[b, e, sl] + bm16
                    sig = 1.0 / (1.0 + jnp.exp(-t))
                    rows_v[b, e, sl] = v16 * sig
                return c2

            lax.fori_loop(0, K, edge_e, 0, unroll=False)

        run_phase(srcg_hbm, dstg_hbm, g2r_starts, g2r_waits, g2r_compute)

        def r2r_starts(b):
            pltpu.async_copy(m_hbm.at[src_v.at[b]], rows_v.at[b], sem_g.at[b])

        def r2r_waits(b):
            pltpu.make_async_copy(m_hbm.at[src_v.at[b]], rows_v.at[b],
                                  sem_g.at[b]).wait()

        run_phase(srcr_hbm, dstr_hbm, r2r_starts, r2r_waits, None)

        plsc.subcore_barrier()

        @pl.when(s < NS - 1)
        def _():
            pltpu.sync_copy(acc.at[pl.ds(row0, ROWS_A)],
                            out_hbm.at[c, pl.ds(row0, ROWS_A)])

        @pl.when(s == NS - 1)
        def _():
            pltpu.sync_copy(acc.at[pl.ds(row0, ROWS_LAST)],
                            out_hbm.at[c, pl.ds(row0, ROWS_LAST)])

    return edge_kernel


_edge_kernel = _make_edge_kernel()


def kernel(x_G, x_R, edge_index_G_to_R, edge_index_R_to_R, W_A, W_B, W_V,
           W_S, W_self, W_msg, ln_g, ln_b, W1, b1, w2):
    xg = x_G[0]
    xr = x_R[0]
    srcg = edge_index_G_to_R[0].astype(jnp.int32)
    dstg = edge_index_G_to_R[1].astype(jnp.int32)
    srcr = edge_index_R_to_R[0].astype(jnp.int32)
    dstr = edge_index_R_to_R[1].astype(jnp.int32)
    zeros = jnp.zeros((ROWS_LAST, RE), _F32)

    bm0, v0, bm1, v1 = pl.pallas_call(
        _pre_body,
        out_shape=[jax.ShapeDtypeStruct((NG, RE), jnp.bfloat16)] * 4,
    )(xg, W_B, W_V)
    bmv0 = lax.bitcast_convert_type(jnp.stack([bm0, v0], axis=-1), _F32)
    bmv1 = lax.bitcast_convert_type(jnp.stack([bm1, v1], axis=-1), _F32)

    a0, m0, sc0 = pl.pallas_call(
        _dense0_body,
        out_shape=[_sds((NR, RE))] * 3,
    )(xr, W_A[0], W_msg[0], W_S[0], W_self[0])

    part0 = _edge_kernel(a0, bmv0, m0, srcg, dstg, srcr, dstr, zeros)

    a1, m1, sc1 = pl.pallas_call(
        _fuse_body,
        out_shape=[_sds((NR, RE))] * 3,
    )(sc0, part0, ln_g[0], ln_b[0], W_A[1], W_msg[1], W_S[1], W_self[1])

    part1 = _edge_kernel(a1, bmv1, m1, srcg, dstg, srcr, dstr, zeros)

    out = pl.pallas_call(
        _head_body,
        out_shape=_sds((NR, 1)),
    )(sc1, part1, ln_g[1], ln_b[1], W1, b1, w2)

    return out.reshape(1, NR)
